# baseline (speedup 1.0000x reference)
import functools

import jax
import jax.numpy as jnp
from jax import lax
from jax.experimental import pallas as pl
from jax.experimental.pallas import tpu as pltpu
from jax.experimental.pallas import tpu_sc as plsc

B = 16384
L = 50
D = 32
OUT = 5

NW = 32
ROWS_PER_W = B // NW
PAIRS_PER_W = ROWS_PER_W // 2
IDX_W = 2 * L + 4


NBUF = 8


def _sc_sums(idx_raw, table):
    mesh = plsc.VectorSubcoreMesh(core_axis_name="c", subcore_axis_name="s")

    @functools.partial(
        pl.kernel,
        mesh=mesh,
        out_type=jax.ShapeDtypeStruct((B, D), jnp.float32),
        scratch_types=[
            pltpu.VMEM((ROWS_PER_W, L), jnp.int32),
            pltpu.VMEM((PAIRS_PER_W, IDX_W), jnp.int32),
            [pltpu.VMEM((IDX_W, D), jnp.float32) for _ in range(NBUF)],
            pltpu.VMEM((ROWS_PER_W, D), jnp.float32),
            [pltpu.SemaphoreType.DMA for _ in range(NBUF)],
        ],
        compiler_params=pltpu.CompilerParams(use_tc_tiling_on_sc=False),
    )
    def k(idx_hbm, table_hbm, out_hbm, idx_s, idx_v, bufs, out_v, sems):
        w = lax.axis_index("s") * 2 + lax.axis_index("c")
        pltpu.sync_copy(idx_hbm.at[pl.ds(w * ROWS_PER_W, ROWS_PER_W)], idx_s)

        zero = jnp.zeros((16,), jnp.int32)

        def repack(p, carry):
            idx_v[p, pl.ds(88, 16)] = zero
            for r in range(2):
                row = 2 * p + r
                dst = r * L
                for off in (0, 16, 32, 34):
                    idx_v[p, pl.ds(dst + off, 16)] = idx_s[row, pl.ds(off, 16)]
            return carry

        lax.fori_loop(0, PAIRS_PER_W, repack, 0)

        for b in range(NBUF):
            pltpu.async_copy(table_hbm.at[idx_v.at[b]], bufs[b], sems[b])

        def accum(buf, g):
            for r in range(2):
                a0 = buf[r * L, pl.ds(0, 16)]
                a1 = buf[r * L, pl.ds(16, 16)]
                for j in range(1, L):
                    a0 = a0 + buf[r * L + j, pl.ds(0, 16)]
                    a1 = a1 + buf[r * L + j, pl.ds(16, 16)]
                out_v[2 * g + r, pl.ds(0, 16)] = a0
                out_v[2 * g + r, pl.ds(16, 16)] = a1

        def body(i, carry):
            for b in range(NBUF):
                g = NBUF * i + b
                pltpu.make_async_copy(
                    table_hbm.at[idx_v.at[g]], bufs[b], sems[b]).wait()
                accum(bufs[b], g)

                @pl.when(g + NBUF < PAIRS_PER_W)
                def _():
                    pltpu.async_copy(
                        table_hbm.at[idx_v.at[g + NBUF]], bufs[b], sems[b])
            return carry

        lax.fori_loop(0, PAIRS_PER_W // NBUF, body, 0)
        pltpu.sync_copy(out_v, out_hbm.at[pl.ds(w * ROWS_PER_W, ROWS_PER_W)])

    return k(idx_raw, table)


def _project(sums, wt_pad, c_pad):
    blk = 2048

    def pk(x_ref, w_ref, c_ref, o_ref):
        o_ref[...] = (
            jnp.dot(x_ref[...], w_ref[...], preferred_element_type=jnp.float32)
            + c_ref[...]
        )

    return pl.pallas_call(
        pk,
        grid=(B // blk,),
        in_specs=[
            pl.BlockSpec((blk, D), lambda i: (i, 0)),
            pl.BlockSpec((D, 8), lambda i: (0, 0)),
            pl.BlockSpec((1, 8), lambda i: (0, 0)),
        ],
        out_specs=pl.BlockSpec((blk, 8), lambda i: (i, 0)),
        out_shape=jax.ShapeDtypeStruct((B, 8), jnp.float32),
    )(sums, wt_pad, c_pad)


def kernel(inputs, embed_table, W, b_l1, bias):
    sums = _sc_sums(inputs.astype(jnp.int32), embed_table)
    wt_pad = jnp.pad(W.T, ((0, 0), (0, 8 - OUT)))
    c_pad = jnp.pad(L * b_l1 + bias, (0, 8 - OUT)).reshape(1, 8)
    return _project(sums, wt_pad, c_pad)[:, :OUT]


# speedup vs baseline: 1.3274x; 1.3274x over previous
import functools

import jax
import jax.numpy as jnp
from jax import lax
from jax.experimental import pallas as pl
from jax.experimental.pallas import tpu as pltpu
from jax.experimental.pallas import tpu_sc as plsc

B = 16384
L = 50
D = 32
OUT = 5
VOCAB = 1000000

VCH = 8192
QTR = VCH // 4
QTR_SH = 11
TABLE_ROWS = ((VOCAB + VCH - 1) // VCH) * VCH

NW = 32
ROWS_PER_W = B // NW
PAIRS_PER_W = ROWS_PER_W // 2
IDX_W = 2 * L + 4


NBUF = 8


def _sc_sums(idx_raw, table):
    mesh = plsc.VectorSubcoreMesh(core_axis_name="c", subcore_axis_name="s")

    @functools.partial(
        pl.kernel,
        mesh=mesh,
        out_type=jax.ShapeDtypeStruct((B, D), jnp.float32),
        name="cbow_gather_sum",
        scratch_types=[
            pltpu.VMEM((ROWS_PER_W, L), jnp.int32),
            pltpu.VMEM((PAIRS_PER_W, IDX_W), jnp.int32),
            [pltpu.VMEM((IDX_W, D), jnp.float32) for _ in range(NBUF)],
            pltpu.VMEM((ROWS_PER_W, D), jnp.float32),
            [pltpu.SemaphoreType.DMA for _ in range(NBUF)],
        ],
        compiler_params=pltpu.CompilerParams(use_tc_tiling_on_sc=False),
    )
    def k(idx_hbm, table_hbm, out_hbm, idx_s, idx_v, bufs, out_v, sems):
        w = lax.axis_index("s") * 2 + lax.axis_index("c")
        pltpu.sync_copy(idx_hbm.at[pl.ds(w * ROWS_PER_W, ROWS_PER_W)], idx_s)

        zero = jnp.zeros((16,), jnp.int32)

        def lrow(v):
            return ((v & jnp.int32(-VCH))
                    + ((v & jnp.int32(QTR - 1)) << 2)
                    + ((v & jnp.int32(VCH - 1)) >> QTR_SH))

        def repack(p, carry):
            idx_v[p, pl.ds(88, 16)] = zero
            for r in range(2):
                row = 2 * p + r
                dst = r * L
                for off in (0, 16, 32, 34):
                    idx_v[p, pl.ds(dst + off, 16)] = lrow(
                        idx_s[row, pl.ds(off, 16)])
            return carry

        lax.fori_loop(0, PAIRS_PER_W, repack, 0)

        for b in range(NBUF):
            pltpu.async_copy(table_hbm.at[idx_v.at[b]], bufs[b], sems[b])

        def accum(buf, g):
            for r in range(2):
                a0 = buf[r * L, pl.ds(0, 16)]
                a1 = buf[r * L, pl.ds(16, 16)]
                for j in range(1, L):
                    a0 = a0 + buf[r * L + j, pl.ds(0, 16)]
                    a1 = a1 + buf[r * L + j, pl.ds(16, 16)]
                out_v[2 * g + r, pl.ds(0, 16)] = a0
                out_v[2 * g + r, pl.ds(16, 16)] = a1

        def body(i, carry):
            for b in range(NBUF):
                g = NBUF * i + b
                pltpu.make_async_copy(
                    table_hbm.at[idx_v.at[g]], bufs[b], sems[b]).wait()
                accum(bufs[b], g)

                @pl.when(g + NBUF < PAIRS_PER_W)
                def _():
                    pltpu.async_copy(
                        table_hbm.at[idx_v.at[g + NBUF]], bufs[b], sems[b])
            return carry

        lax.fori_loop(0, PAIRS_PER_W // NBUF, body, 0)
        pltpu.sync_copy(out_v, out_hbm.at[pl.ds(w * ROWS_PER_W, ROWS_PER_W)])

    return k(idx_raw, table)


def _repack_table(embed_table):
    p = jnp.transpose(embed_table)
    grid = pl.cdiv(VOCAB, VCH)

    def rk(p_ref, r_ref):
        x = p_ref[...]
        for q in range(4):
            r_ref[:, D * q:D * (q + 1)] = jnp.transpose(
                x[:, QTR * q:QTR * (q + 1)])

    r = pl.pallas_call(
        rk,
        grid=(grid,),
        in_specs=[pl.BlockSpec((D, VCH), lambda i: (0, i))],
        out_specs=pl.BlockSpec((QTR, 128), lambda i: (i, 0)),
        out_shape=jax.ShapeDtypeStruct((TABLE_ROWS * D // 128, 128),
                                       jnp.float32),
    )(p)
    return r.reshape(TABLE_ROWS, D)


def _project(sums, wt_pad, c_pad):
    blk = 2048

    def pk(x_ref, w_ref, c_ref, o_ref):
        o_ref[...] = (
            jnp.dot(x_ref[...], w_ref[...], preferred_element_type=jnp.float32)
            + c_ref[...]
        )

    return pl.pallas_call(
        pk,
        grid=(B // blk,),
        in_specs=[
            pl.BlockSpec((blk, D), lambda i: (i, 0)),
            pl.BlockSpec((D, 8), lambda i: (0, 0)),
            pl.BlockSpec((1, 8), lambda i: (0, 0)),
        ],
        out_specs=pl.BlockSpec((blk, 8), lambda i: (i, 0)),
        out_shape=jax.ShapeDtypeStruct((B, 8), jnp.float32),
    )(sums, wt_pad, c_pad)


def kernel(inputs, embed_table, W, b_l1, bias):
    table_lin = _repack_table(embed_table)
    sums = _sc_sums(inputs.astype(jnp.int32), table_lin)
    wt_pad = jnp.pad(W.T, ((0, 0), (0, 8 - OUT)))
    c_pad = jnp.pad(L * b_l1 + bias, (0, 8 - OUT)).reshape(1, 8)
    return _project(sums, wt_pad, c_pad)[:, :OUT]


# speedup vs baseline: 1.7299x; 1.3032x over previous
import functools

import jax
import jax.numpy as jnp
from jax import lax
from jax.experimental import pallas as pl
from jax.experimental.pallas import tpu as pltpu
from jax.experimental.pallas import tpu_sc as plsc

B = 16384
L = 50
D = 32
OUT = 5
VOCAB = 1000000

VCH = 8192
ECH = VCH // 8
ECH_SH = 10
TABLE_ROWS = ((VOCAB + VCH - 1) // VCH) * VCH

NW = 32
ROWS_PER_W = B // NW
PAIRS_PER_W = ROWS_PER_W // 2
IDX_W = 2 * L + 4


NBUF = 8


def _sc_sums(idx_raw, table):
    mesh = plsc.VectorSubcoreMesh(core_axis_name="c", subcore_axis_name="s")

    @functools.partial(
        pl.kernel,
        mesh=mesh,
        out_type=jax.ShapeDtypeStruct((B, D), jnp.float32),
        name="cbow_gather_sum",
        scratch_types=[
            pltpu.VMEM((ROWS_PER_W, L), jnp.int32),
            pltpu.VMEM((PAIRS_PER_W, IDX_W), jnp.int32),
            [pltpu.VMEM((IDX_W, 16), jnp.int32) for _ in range(NBUF)],
            pltpu.VMEM((ROWS_PER_W, D), jnp.float32),
            [pltpu.SemaphoreType.DMA for _ in range(NBUF)],
        ],
        compiler_params=pltpu.CompilerParams(
            use_tc_tiling_on_sc=False, needs_layout_passes=False),
    )
    def k(idx_hbm, table_hbm, out_hbm, idx_s, idx_v, bufs, out_v, sems):
        w = lax.axis_index("s") * 2 + lax.axis_index("c")
        pltpu.sync_copy(idx_hbm.at[pl.ds(w * ROWS_PER_W, ROWS_PER_W)], idx_s)

        zero = jnp.zeros((16,), jnp.int32)

        def lrow(v):
            return ((v & jnp.int32(-VCH))
                    + ((v & jnp.int32(ECH - 1)) << 3)
                    + ((v & jnp.int32(VCH - 1)) >> ECH_SH))

        def repack(p, carry):
            idx_v[p, pl.ds(88, 16)] = zero
            for r in range(2):
                row = 2 * p + r
                dst = r * L
                for off in (0, 16, 32, 34):
                    idx_v[p, pl.ds(dst + off, 16)] = lrow(
                        idx_s[row, pl.ds(off, 16)])
            return carry

        lax.fori_loop(0, PAIRS_PER_W, repack, 0)

        for b in range(NBUF):
            pltpu.async_copy(table_hbm.at[idx_v.at[b]], bufs[b], sems[b])

        def accum(buf, g):
            for r in range(2):
                v = buf[r * L, :]
                a0 = plsc.bitcast(v << 16, jnp.float32)
                a1 = plsc.bitcast(v & jnp.int32(-65536), jnp.float32)
                for j in range(1, L):
                    v = buf[r * L + j, :]
                    a0 = a0 + plsc.bitcast(v << 16, jnp.float32)
                    a1 = a1 + plsc.bitcast(v & jnp.int32(-65536), jnp.float32)
                out_v[2 * g + r, pl.ds(0, 16)] = a0
                out_v[2 * g + r, pl.ds(16, 16)] = a1

        def body(i, carry):
            for b in range(NBUF):
                g = NBUF * i + b
                pltpu.make_async_copy(
                    table_hbm.at[idx_v.at[g]], bufs[b], sems[b]).wait()
                accum(bufs[b], g)

                @pl.when(g + NBUF < PAIRS_PER_W)
                def _():
                    pltpu.async_copy(
                        table_hbm.at[idx_v.at[g + NBUF]], bufs[b], sems[b])
            return carry

        lax.fori_loop(0, PAIRS_PER_W // NBUF, body, 0)
        pltpu.sync_copy(out_v, out_hbm.at[pl.ds(w * ROWS_PER_W, ROWS_PER_W)])

    return k(idx_raw, table)


def _repack_table(embed_table):
    p = jnp.transpose(embed_table)
    grid = pl.cdiv(VOCAB, VCH)

    def bf16_hi(t):
        return (t + jnp.int32(0x7FFF) + ((t >> 16) & 1)) & jnp.int32(-65536)

    def rk(p_ref, r_ref):
        x = p_ref[...]
        ta = jax.lax.bitcast_convert_type(x[:16, :], jnp.int32)
        tb = jax.lax.bitcast_convert_type(x[16:, :], jnp.int32)
        lo = (bf16_hi(ta) >> 16) & jnp.int32(0xFFFF)
        words = bf16_hi(tb) | lo
        for e in range(8):
            r_ref[:, 16 * e:16 * (e + 1)] = jnp.transpose(
                words[:, ECH * e:ECH * (e + 1)])

    r = pl.pallas_call(
        rk,
        grid=(grid,),
        in_specs=[pl.BlockSpec((D, VCH), lambda i: (0, i))],
        out_specs=pl.BlockSpec((ECH, 128), lambda i: (i, 0)),
        out_shape=jax.ShapeDtypeStruct((TABLE_ROWS * 16 // 128, 128),
                                       jnp.int32),
    )(p)
    return r.reshape(TABLE_ROWS, 16)


def _project(sums, wt_pad, c_pad):
    blk = 2048

    def pk(x_ref, w_ref, c_ref, o_ref):
        o_ref[...] = (
            jnp.dot(x_ref[...], w_ref[...], preferred_element_type=jnp.float32)
            + c_ref[...]
        )

    return pl.pallas_call(
        pk,
        grid=(B // blk,),
        in_specs=[
            pl.BlockSpec((blk, D), lambda i: (i, 0)),
            pl.BlockSpec((D, 8), lambda i: (0, 0)),
            pl.BlockSpec((1, 8), lambda i: (0, 0)),
        ],
        out_specs=pl.BlockSpec((blk, 8), lambda i: (i, 0)),
        out_shape=jax.ShapeDtypeStruct((B, 8), jnp.float32),
    )(sums, wt_pad, c_pad)


def kernel(inputs, embed_table, W, b_l1, bias):
    table_lin = _repack_table(embed_table)
    sums = _sc_sums(inputs.astype(jnp.int32), table_lin)
    wt_pad = jnp.pad(W.T, ((0, 0), (0, 8 - OUT)))
    c_pad = jnp.pad(L * b_l1 + bias, (0, 8 - OUT)).reshape(1, 8)
    return _project(sums, wt_pad, c_pad)[:, :OUT]


# speedup vs baseline: 1.7425x; 1.0073x over previous
import functools

import jax
import jax.numpy as jnp
from jax import lax
from jax.experimental import pallas as pl
from jax.experimental.pallas import tpu as pltpu
from jax.experimental.pallas import tpu_sc as plsc

B = 16384
L = 50
D = 32
OUT = 5
VOCAB = 1000000

VCH = 16384
ECH = VCH // 8
ECH_SH = ECH.bit_length() - 1
TABLE_ROWS = ((VOCAB + VCH - 1) // VCH) * VCH

NW = 32
ROWS_PER_W = B // NW
PAIRS_PER_W = ROWS_PER_W // 2
IDX_W = 2 * L + 4


NBUF = 8


def _sc_sums(idx_raw, table):
    mesh = plsc.VectorSubcoreMesh(core_axis_name="c", subcore_axis_name="s")

    @functools.partial(
        pl.kernel,
        mesh=mesh,
        out_type=jax.ShapeDtypeStruct((B, D), jnp.float32),
        name="cbow_gather_sum",
        scratch_types=[
            pltpu.VMEM((ROWS_PER_W, L), jnp.int32),
            pltpu.VMEM((PAIRS_PER_W, IDX_W), jnp.int32),
            [pltpu.VMEM((IDX_W, 16), jnp.int32) for _ in range(NBUF)],
            pltpu.VMEM((ROWS_PER_W, D), jnp.float32),
            [pltpu.SemaphoreType.DMA for _ in range(NBUF)],
        ],
        compiler_params=pltpu.CompilerParams(
            use_tc_tiling_on_sc=False, needs_layout_passes=False),
    )
    def k(idx_hbm, table_hbm, out_hbm, idx_s, idx_v, bufs, out_v, sems):
        w = lax.axis_index("s") * 2 + lax.axis_index("c")
        pltpu.sync_copy(idx_hbm.at[pl.ds(w * ROWS_PER_W, ROWS_PER_W)], idx_s)

        zero = jnp.zeros((16,), jnp.int32)

        def lrow(v):
            return ((v & jnp.int32(-VCH))
                    + ((v & jnp.int32(ECH - 1)) << 3)
                    + ((v & jnp.int32(VCH - 1)) >> ECH_SH))

        def repack(p, carry):
            idx_v[p, pl.ds(88, 16)] = zero
            for r in range(2):
                row = 2 * p + r
                dst = r * L
                for off in (0, 16, 32, 34):
                    idx_v[p, pl.ds(dst + off, 16)] = lrow(
                        idx_s[row, pl.ds(off, 16)])
            return carry

        lax.fori_loop(0, PAIRS_PER_W, repack, 0)

        for b in range(NBUF):
            pltpu.async_copy(table_hbm.at[idx_v.at[b]], bufs[b], sems[b])

        def accum(buf, g):
            for r in range(2):
                v = buf[r * L, :]
                a0 = plsc.bitcast(v << 16, jnp.float32)
                a1 = plsc.bitcast(v & jnp.int32(-65536), jnp.float32)
                for j in range(1, L):
                    v = buf[r * L + j, :]
                    a0 = a0 + plsc.bitcast(v << 16, jnp.float32)
                    a1 = a1 + plsc.bitcast(v & jnp.int32(-65536), jnp.float32)
                out_v[2 * g + r, pl.ds(0, 16)] = a0
                out_v[2 * g + r, pl.ds(16, 16)] = a1

        def body(i, carry):
            for b in range(NBUF):
                g = NBUF * i + b
                pltpu.make_async_copy(
                    table_hbm.at[idx_v.at[g]], bufs[b], sems[b]).wait()
                accum(bufs[b], g)

                @pl.when(g + NBUF < PAIRS_PER_W)
                def _():
                    pltpu.async_copy(
                        table_hbm.at[idx_v.at[g + NBUF]], bufs[b], sems[b])
            return carry

        lax.fori_loop(0, PAIRS_PER_W // NBUF, body, 0)
        pltpu.sync_copy(out_v, out_hbm.at[pl.ds(w * ROWS_PER_W, ROWS_PER_W)])

    return k(idx_raw, table)


def _repack_table(embed_table):
    p = jnp.transpose(embed_table)
    grid = pl.cdiv(VOCAB, VCH)

    def bf16_hi(t):
        return (t + jnp.int32(0x7FFF) + ((t >> 16) & 1)) & jnp.int32(-65536)

    def rk(p_ref, r_ref):
        x = p_ref[...]
        ta = jax.lax.bitcast_convert_type(x[:16, :], jnp.int32)
        tb = jax.lax.bitcast_convert_type(x[16:, :], jnp.int32)
        lo = (bf16_hi(ta) >> 16) & jnp.int32(0xFFFF)
        words = bf16_hi(tb) | lo
        for e in range(8):
            r_ref[:, 16 * e:16 * (e + 1)] = jnp.transpose(
                words[:, ECH * e:ECH * (e + 1)])

    r = pl.pallas_call(
        rk,
        grid=(grid,),
        in_specs=[pl.BlockSpec((D, VCH), lambda i: (0, i))],
        out_specs=pl.BlockSpec((ECH, 128), lambda i: (i, 0)),
        out_shape=jax.ShapeDtypeStruct((TABLE_ROWS * 16 // 128, 128),
                                       jnp.int32),
    )(p)
    return r.reshape(TABLE_ROWS, 16)


def _project(sums, wt_pad, c_pad):
    blk = 2048

    def pk(x_ref, w_ref, c_ref, o_ref):
        o_ref[...] = (
            jnp.dot(x_ref[...], w_ref[...], preferred_element_type=jnp.float32)
            + c_ref[...]
        )

    return pl.pallas_call(
        pk,
        grid=(B // blk,),
        in_specs=[
            pl.BlockSpec((blk, D), lambda i: (i, 0)),
            pl.BlockSpec((D, 8), lambda i: (0, 0)),
            pl.BlockSpec((1, 8), lambda i: (0, 0)),
        ],
        out_specs=pl.BlockSpec((blk, 8), lambda i: (i, 0)),
        out_shape=jax.ShapeDtypeStruct((B, 8), jnp.float32),
    )(sums, wt_pad, c_pad)


def kernel(inputs, embed_table, W, b_l1, bias):
    table_lin = _repack_table(embed_table)
    sums = _sc_sums(inputs.astype(jnp.int32), table_lin)
    wt_pad = jnp.pad(W.T, ((0, 0), (0, 8 - OUT)))
    c_pad = jnp.pad(L * b_l1 + bias, (0, 8 - OUT)).reshape(1, 8)
    return _project(sums, wt_pad, c_pad)[:, :OUT]
